# baseline (speedup 1.0000x reference)
import functools
import math

import jax
import jax.numpy as jnp
from jax import lax
from jax.experimental import pallas as pl
from jax.experimental.pallas import tpu as pltpu
from jax.experimental.pallas import tpu_sc as plsc

_NC = 2
_NS = 16
_NW = _NC * _NS
_BLK = 128
_SBLK = 8


def _prescale_pad(table, scale):
    v, d = table.shape
    blk = 5000
    assert v % blk == 0

    def body(t_ref, o_ref):
        o_ref[:, :d] = t_ref[...] * scale
        o_ref[:, d:] = jnp.zeros((blk, 128 - d), jnp.float32)

    return pl.pallas_call(
        body,
        grid=(v // blk,),
        in_specs=[pl.BlockSpec((blk, d), lambda i: (i, 0))],
        out_specs=pl.BlockSpec((blk, 128), lambda i: (i, 0)),
        out_shape=jax.ShapeDtypeStruct((v, 128), jnp.float32),
    )(table)


def _gather_t(scaled, idx_t, b, s, d):
    assert b == _BLK * _NW
    n_sblk = s // _SBLK
    n_vregs = _BLK // 16
    mesh = plsc.VectorSubcoreMesh(core_axis_name="c", subcore_axis_name="s")

    @functools.partial(
        pl.kernel,
        out_type=jax.ShapeDtypeStruct((s, d, b), jnp.float32),
        mesh=mesh,
        compiler_params=pltpu.CompilerParams(needs_layout_passes=False),
        scratch_types=[
            pltpu.VMEM((_SBLK, _BLK), jnp.int32),
            pltpu.VMEM((_BLK, 128), jnp.float32),
            pltpu.VMEM((_BLK, 128), jnp.float32),
            pltpu.VMEM((d, _BLK), jnp.float32),
            pltpu.VMEM((d, _BLK), jnp.float32),
            pltpu.SemaphoreType.DMA,
            pltpu.SemaphoreType.DMA,
            pltpu.SemaphoreType.DMA,
            pltpu.SemaphoreType.DMA,
        ],
    )
    def gather(table_hbm, idx_hbm, out_hbm, idx_v, rv0, rv1, tv0, tv1,
               sg0, sg1, so0, so1):
        wid = lax.axis_index("s") * _NC + lax.axis_index("c")
        b0 = wid * _BLK
        rv = (rv0, rv1)
        tv = (tv0, tv1)
        sg = (sg0, sg1)
        so = (so0, so1)

        def transpose_block(src, dst):
            lane = lax.broadcasted_iota(jnp.int32, (16,), 0)

            def erow(e, carry):
                for c in range(n_vregs):
                    rows = lane + (16 * c)
                    cols = jnp.broadcast_to(e, (16,))
                    dst[e, pl.ds(16 * c, 16)] = plsc.load_gather(
                        src, [rows, cols])
                return carry

            lax.fori_loop(0, d, erow, 0)

        def sblk_body(g, carry):
            s0 = g * _SBLK
            pltpu.sync_copy(
                idx_hbm.at[pl.ds(s0, _SBLK), pl.ds(b0, _BLK)], idx_v)
            gather_cp = [None] * _SBLK
            out_cp = [None] * _SBLK
            gather_cp[0] = pltpu.async_copy(
                table_hbm.at[idx_v.at[0]], rv[0], sg[0])
            for q in range(_SBLK):
                p = q % 2
                if q + 1 < _SBLK:
                    gather_cp[q + 1] = pltpu.async_copy(
                        table_hbm.at[idx_v.at[q + 1]], rv[1 - p], sg[1 - p])
                gather_cp[q].wait()
                if q >= 2:
                    out_cp[q - 2].wait()
                transpose_block(rv[p], tv[p])
                out_cp[q] = pltpu.async_copy(
                    tv[p],
                    out_hbm.at[s0 + q, :, pl.ds(b0, _BLK)],
                    so[p])
            out_cp[_SBLK - 2].wait()
            out_cp[_SBLK - 1].wait()
            return carry

        lax.fori_loop(0, n_sblk, sblk_body, 0)

    return gather(scaled, idx_t)


def kernel(tokens, table):
    b, s = tokens.shape
    v, d = table.shape
    scale = math.sqrt(float(d))
    scaled = _prescale_pad(table, scale)
    idx_t = tokens.T.astype(jnp.int32)
    out_t = _gather_t(scaled, idx_t, b, s, d)
    return jnp.transpose(out_t, (2, 0, 1))


# speedup vs baseline: 1.2006x; 1.2006x over previous
import functools
import math

import jax
import jax.numpy as jnp
from jax import lax
from jax.experimental import pallas as pl
from jax.experimental.pallas import tpu as pltpu
from jax.experimental.pallas import tpu_sc as plsc

_NC = 2
_NS = 16
_NW = _NC * _NS
_BLK = 128
_SBLK = 8


def _prescale_pad(table, scale):
    v, d = table.shape
    blk = 5000
    assert v % blk == 0

    def body(t_ref, o_ref):
        o_ref[:, :d] = t_ref[...] * scale
        o_ref[:, d:] = jnp.zeros((blk, 128 - d), jnp.float32)

    return pl.pallas_call(
        body,
        grid=(v // blk,),
        in_specs=[pl.BlockSpec((blk, d), lambda i: (i, 0))],
        out_specs=pl.BlockSpec((blk, 128), lambda i: (i, 0)),
        out_shape=jax.ShapeDtypeStruct((v, 128), jnp.float32),
    )(table)


def _gather_t(scaled, idx_t, b, s, d):
    assert b == _BLK * _NW
    n_sblk = s // _SBLK
    n_vregs = _BLK // 16
    mesh = plsc.VectorSubcoreMesh(core_axis_name="c", subcore_axis_name="s")

    @functools.partial(
        pl.kernel,
        out_type=jax.ShapeDtypeStruct((s, d, b), jnp.float32),
        mesh=mesh,
        compiler_params=pltpu.CompilerParams(needs_layout_passes=False),
        scratch_types=[
            pltpu.VMEM((_SBLK, _BLK), jnp.int32),
            pltpu.VMEM((_BLK, 128), jnp.float32),
            pltpu.VMEM((_BLK, 128), jnp.float32),
            pltpu.VMEM((d, _BLK), jnp.float32),
            pltpu.VMEM((d, _BLK), jnp.float32),
            pltpu.SemaphoreType.DMA,
            pltpu.SemaphoreType.DMA,
            pltpu.SemaphoreType.DMA,
            pltpu.SemaphoreType.DMA,
        ],
    )
    def gather(table_hbm, idx_hbm, out_hbm, idx_v, rv0, rv1, tv0, tv1,
               sg0, sg1, so0, so1):
        wid = lax.axis_index("s") * _NC + lax.axis_index("c")
        b0 = wid * _BLK
        rv = (rv0, rv1)
        tv = (tv0, tv1)
        sg = (sg0, sg1)
        so = (so0, so1)

        lane = lax.broadcasted_iota(jnp.int32, (16,), 0)
        erow_vecs = [lane + (16 * c) for c in range(d // 16)]

        def transpose_block(src, dst):
            def trow(r, carry):
                cols = jnp.broadcast_to(r, (16,))
                for c in range(d // 16):
                    vals = src[r, pl.ds(16 * c, 16)]
                    plsc.store_scatter(dst, [erow_vecs[c], cols], vals)
                return carry

            lax.fori_loop(0, _BLK, trow, 0)

        def sblk_body(g, carry):
            s0 = g * _SBLK
            pltpu.sync_copy(
                idx_hbm.at[pl.ds(s0, _SBLK), pl.ds(b0, _BLK)], idx_v)
            gather_cp = [None] * _SBLK
            out_cp = [None] * _SBLK
            gather_cp[0] = pltpu.async_copy(
                table_hbm.at[idx_v.at[0]], rv[0], sg[0])
            for q in range(_SBLK):
                p = q % 2
                if q + 1 < _SBLK:
                    gather_cp[q + 1] = pltpu.async_copy(
                        table_hbm.at[idx_v.at[q + 1]], rv[1 - p], sg[1 - p])
                gather_cp[q].wait()
                if q >= 2:
                    out_cp[q - 2].wait()
                transpose_block(rv[p], tv[p])
                out_cp[q] = pltpu.async_copy(
                    tv[p],
                    out_hbm.at[s0 + q, :, pl.ds(b0, _BLK)],
                    so[p])
            out_cp[_SBLK - 2].wait()
            out_cp[_SBLK - 1].wait()
            return carry

        lax.fori_loop(0, n_sblk, sblk_body, 0)

    return gather(scaled, idx_t)


def kernel(tokens, table):
    b, s = tokens.shape
    v, d = table.shape
    scale = math.sqrt(float(d))
    scaled = _prescale_pad(table, scale)
    idx_t = tokens.T.astype(jnp.int32)
    out_t = _gather_t(scaled, idx_t, b, s, d)
    return jnp.transpose(out_t, (2, 0, 1))


# speedup vs baseline: 2.4098x; 2.0072x over previous
import functools
import math

import jax
import jax.numpy as jnp
from jax import lax
from jax.experimental import pallas as pl
from jax.experimental.pallas import tpu as pltpu
from jax.experimental.pallas import tpu_sc as plsc

_NC = 2
_NS = 16
_NW = _NC * _NS

_CHUNK = 128
_GROUP = 1024
_CPG = _GROUP // _CHUNK


def _prescale_pad(table, scale):
    v, d = table.shape
    blk = 5000
    assert v % blk == 0

    def body(t_ref, o_ref):
        o_ref[:, :d] = t_ref[...] * scale
        o_ref[:, d:] = jnp.zeros((blk, 128 - d), jnp.float32)

    return pl.pallas_call(
        body,
        grid=(v // blk,),
        in_specs=[pl.BlockSpec((blk, d), lambda i: (i, 0))],
        out_specs=pl.BlockSpec((blk, 128), lambda i: (i, 0)),
        out_shape=jax.ShapeDtypeStruct((v, 128), jnp.float32),
    )(table)


def _gather(scaled, idx2d, n, d):
    n_per_w = n // _NW
    n_groups = n_per_w // _GROUP
    n_vregs = d // 16
    mesh = plsc.VectorSubcoreMesh(core_axis_name="c", subcore_axis_name="s")

    @functools.partial(
        pl.kernel,
        out_type=jax.ShapeDtypeStruct((n, d), jnp.float32),
        mesh=mesh,
        scratch_types=[
            pltpu.VMEM((8, 128), jnp.int32),
            pltpu.VMEM((_CHUNK, 128), jnp.float32),
            pltpu.VMEM((_CHUNK, 128), jnp.float32),
            pltpu.VMEM((_CHUNK, 128), jnp.float32),
            pltpu.VMEM((_CHUNK, d), jnp.float32),
            pltpu.VMEM((_CHUNK, d), jnp.float32),
            pltpu.SemaphoreType.DMA,
            pltpu.SemaphoreType.DMA,
            pltpu.SemaphoreType.DMA,
            pltpu.SemaphoreType.DMA,
            pltpu.SemaphoreType.DMA,
        ],
    )
    def gather(table_hbm, idx_hbm, out_hbm, idx_v, rv0, rv1, rv2, cv0, cv1,
               sg0, sg1, sg2, so0, so1):
        wid = lax.axis_index("s") * _NC + lax.axis_index("c")
        base = wid * n_per_w
        idx_base = wid * (n_per_w // 128)
        rv = (rv0, rv1, rv2)
        cv = (cv0, cv1)
        sg = (sg0, sg1, sg2)
        so = (so0, so1)

        def compact(src, dst):
            def crow(i, carry):
                r0 = i * 8
                for k in range(8):
                    for j in range(n_vregs):
                        dst[r0 + k, pl.ds(j * 16, 16)] = (
                            src[r0 + k, pl.ds(j * 16, 16)])
                return carry
            lax.fori_loop(0, _CHUNK // 8, crow, 0)

        def group_body(g, carry):
            pltpu.sync_copy(idx_hbm.at[pl.ds(idx_base + g * 8, 8)], idx_v)
            gather_cp = [None] * _CPG
            out_cp = [None] * _CPG
            gather_cp[0] = pltpu.async_copy(
                table_hbm.at[idx_v.at[0]], rv[0], sg[0])
            gather_cp[1] = pltpu.async_copy(
                table_hbm.at[idx_v.at[1]], rv[1], sg[1])
            for q in range(_CPG):
                p = q % 2
                r3 = q % 3
                if q + 2 < _CPG:
                    gather_cp[q + 2] = pltpu.async_copy(
                        table_hbm.at[idx_v.at[q + 2]],
                        rv[(q + 2) % 3], sg[(q + 2) % 3])
                gather_cp[q].wait()
                if q >= 2:
                    out_cp[q - 2].wait()
                compact(rv[r3], cv[p])
                out_cp[q] = pltpu.async_copy(
                    cv[p],
                    out_hbm.at[pl.ds(base + g * _GROUP + q * _CHUNK, _CHUNK)],
                    so[p])
            out_cp[_CPG - 2].wait()
            out_cp[_CPG - 1].wait()
            return carry

        lax.fori_loop(0, n_groups, group_body, 0)

    return gather(scaled, idx2d)


def kernel(tokens, table):
    b, s = tokens.shape
    v, d = table.shape
    n = b * s
    scale = math.sqrt(float(d))
    scaled = _prescale_pad(table, scale)
    idx2d = tokens.reshape(n // 128, 128).astype(jnp.int32)
    out = _gather(scaled, idx2d, n, d)
    return out.reshape(b, s, d)


# speedup vs baseline: 2.4177x; 1.0033x over previous
import functools
import math

import jax
import jax.numpy as jnp
from jax import lax
from jax.experimental import pallas as pl
from jax.experimental.pallas import tpu as pltpu
from jax.experimental.pallas import tpu_sc as plsc

_NC = 2
_NS = 16
_NW = _NC * _NS

_CHUNK = 128
_GROUP = 5120
_CPG = _GROUP // _CHUNK


def _prescale_pad(table, scale):
    v, d = table.shape
    blk = 5000
    assert v % blk == 0

    def body(t_ref, o_ref):
        o_ref[:, :d] = t_ref[...] * scale
        o_ref[:, d:] = jnp.zeros((blk, 128 - d), jnp.float32)

    return pl.pallas_call(
        body,
        grid=(v // blk,),
        in_specs=[pl.BlockSpec((blk, d), lambda i: (i, 0))],
        out_specs=pl.BlockSpec((blk, 128), lambda i: (i, 0)),
        out_shape=jax.ShapeDtypeStruct((v, 128), jnp.float32),
    )(table)


def _gather(scaled, idx2d, n, d):
    n_per_w = n // _NW
    n_groups = n_per_w // _GROUP
    n_vregs = d // 16
    mesh = plsc.VectorSubcoreMesh(core_axis_name="c", subcore_axis_name="s")

    @functools.partial(
        pl.kernel,
        out_type=jax.ShapeDtypeStruct((n, d), jnp.float32),
        mesh=mesh,
        scratch_types=[
            pltpu.VMEM((_GROUP // 128, 128), jnp.int32),
            pltpu.VMEM((_CHUNK, 128), jnp.float32),
            pltpu.VMEM((_CHUNK, 128), jnp.float32),
            pltpu.VMEM((_CHUNK, 128), jnp.float32),
            pltpu.VMEM((_CHUNK, d), jnp.float32),
            pltpu.VMEM((_CHUNK, d), jnp.float32),
            pltpu.SemaphoreType.DMA,
            pltpu.SemaphoreType.DMA,
            pltpu.SemaphoreType.DMA,
            pltpu.SemaphoreType.DMA,
            pltpu.SemaphoreType.DMA,
        ],
    )
    def gather(table_hbm, idx_hbm, out_hbm, idx_v, rv0, rv1, rv2, cv0, cv1,
               sg0, sg1, sg2, so0, so1):
        wid = lax.axis_index("s") * _NC + lax.axis_index("c")
        base = wid * n_per_w
        idx_base = wid * (n_per_w // 128)
        rv = (rv0, rv1, rv2)
        cv = (cv0, cv1)
        sg = (sg0, sg1, sg2)
        so = (so0, so1)

        def compact(src, dst):
            def crow(i, carry):
                r0 = i * 8
                for k in range(8):
                    for j in range(n_vregs):
                        dst[r0 + k, pl.ds(j * 16, 16)] = (
                            src[r0 + k, pl.ds(j * 16, 16)])
                return carry
            lax.fori_loop(0, _CHUNK // 8, crow, 0)

        def group_body(g, carry):
            pltpu.sync_copy(idx_hbm.at[pl.ds(idx_base + g * (_GROUP // 128), _GROUP // 128)], idx_v)
            gather_cp = [None] * _CPG
            out_cp = [None] * _CPG
            gather_cp[0] = pltpu.async_copy(
                table_hbm.at[idx_v.at[0]], rv[0], sg[0])
            gather_cp[1] = pltpu.async_copy(
                table_hbm.at[idx_v.at[1]], rv[1], sg[1])
            for q in range(_CPG):
                p = q % 2
                r3 = q % 3
                if q + 2 < _CPG:
                    gather_cp[q + 2] = pltpu.async_copy(
                        table_hbm.at[idx_v.at[q + 2]],
                        rv[(q + 2) % 3], sg[(q + 2) % 3])
                gather_cp[q].wait()
                if q >= 2:
                    out_cp[q - 2].wait()
                compact(rv[r3], cv[p])
                out_cp[q] = pltpu.async_copy(
                    cv[p],
                    out_hbm.at[pl.ds(base + g * _GROUP + q * _CHUNK, _CHUNK)],
                    so[p])
            out_cp[_CPG - 2].wait()
            out_cp[_CPG - 1].wait()
            return carry

        lax.fori_loop(0, n_groups, group_body, 0)

    return gather(scaled, idx2d)


def kernel(tokens, table):
    b, s = tokens.shape
    v, d = table.shape
    n = b * s
    scale = math.sqrt(float(d))
    scaled = _prescale_pad(table, scale)
    idx2d = tokens.reshape(n // 128, 128).astype(jnp.int32)
    out = _gather(scaled, idx2d, n, d)
    return out.reshape(b, s, d)
